# baseline (speedup 1.0000x reference)
import functools

import jax
import jax.numpy as jnp
from jax import lax
from jax.experimental import pallas as pl
from jax.experimental.pallas import tpu as pltpu
from jax.experimental.pallas import tpu_sc as plsc

N = 65536
D_IN = 100
D_HID = 200
LATENT = 256
K = 8192
COMMIT_SCALE = 0.25

BM = 256
PRE_BM = 2048
CHUNKS = ((0, 2736), (2736, 2736), (5472, 2720))

SC_CORES = 2
SC_SUBCORES = 16
SC_WORKERS = SC_CORES * SC_SUBCORES
SC_CHUNK = 128


def _pre_body(c_ref, W1_ref, b1_ref, a_ref):
    a_ref[...] = (jnp.dot(c_ref[...], W1_ref[...],
                          preferred_element_type=jnp.float32)
                  + b1_ref[...][None, :])


def _pre_call(c, W1, b1):
    return pl.pallas_call(
        _pre_body,
        grid=(N // PRE_BM,),
        in_specs=[
            pl.BlockSpec((PRE_BM, D_IN), lambda i: (i, 0)),
            pl.BlockSpec((D_IN, D_HID), lambda i: (0, 0)),
            pl.BlockSpec((D_HID,), lambda i: (0,)),
        ],
        out_specs=pl.BlockSpec((PRE_BM, D_HID), lambda i: (i, 0)),
        out_shape=jax.ShapeDtypeStruct((N, D_HID), jnp.float32),
        compiler_params=pltpu.CompilerParams(
            dimension_semantics=("arbitrary",)),
    )(c, W1, b1)


def _vq_body(h_ref, W2_ref, b2_ref, E_ref, Wmu_ref, bmu_ref,
             Wlv_ref, blv_ref, ze_ref, idx_ref, mu_ref, lv_ref, dsum_ref,
             s_ref):
    i = pl.program_id(0)

    @pl.when(i == 0)
    def _init():
        e = E_ref[...]
        s_ref[...] = jnp.sum(e * e, axis=1)[None, :]
        dsum_ref[...] = jnp.zeros((1, 1), jnp.float32)

    z = jnp.dot(h_ref[...], W2_ref[...], preferred_element_type=jnp.float32) + b2_ref[...][None, :]
    ze_ref[...] = z

    r = jnp.sum(z * z, axis=1, keepdims=True)

    run_v = jnp.full((BM, 1), jnp.inf, dtype=jnp.float32)
    run_f = jnp.full((BM, 1), jnp.inf, dtype=jnp.float32)
    run_i = jnp.zeros((BM, 1), dtype=jnp.int32)
    for off, w in CHUNKS:
        et = E_ref[pl.ds(off, w), :]
        st = s_ref[0, pl.ds(off, w)]
        m = lax.dot_general(z, et, (((1,), (1,)), ((), ())),
                            preferred_element_type=jnp.float32)
        d = (r + st[None, :]) - 2.0 * m
        v = jnp.min(d, axis=1, keepdims=True)
        iota = lax.broadcasted_iota(jnp.int32, (BM, w), 1)
        ti = jnp.min(jnp.where(d == v, iota, K), axis=1, keepdims=True) + off
        better = v < run_v
        run_i = jnp.where(better, ti, run_i)
        run_f = jnp.where(better, v, run_f)
        run_v = (jnp.where(better, v, run_v)
                 .astype(jnp.bfloat16).astype(jnp.float32))

    idx_ref[...] = run_i.reshape(BM)
    dsum_ref[...] += jnp.sum(run_f).reshape(1, 1)

    mu_ref[...] = jnp.dot(z, Wmu_ref[...], preferred_element_type=jnp.float32) + bmu_ref[...][None, :]
    lv_ref[...] = jnp.dot(z, Wlv_ref[...], preferred_element_type=jnp.float32) + blv_ref[...][None, :]


def _vq_call(h, W2, b2, embedding, Wmu, bmu, Wlv, blv, interpret=False):
    whole = lambda *shape: pl.BlockSpec(shape, lambda i: tuple(0 for _ in shape))
    grid = (N // BM,)
    return pl.pallas_call(
        _vq_body,
        grid=grid,
        in_specs=[
            pl.BlockSpec((BM, D_HID), lambda i: (i, 0)),
            whole(D_HID, LATENT),
            whole(LATENT),
            whole(K, LATENT),
            whole(LATENT, LATENT),
            whole(LATENT),
            whole(LATENT, LATENT),
            whole(LATENT),
        ],
        out_specs=[
            pl.BlockSpec((BM, LATENT), lambda i: (i, 0)),
            pl.BlockSpec((BM,), lambda i: (i,)),
            pl.BlockSpec((BM, LATENT), lambda i: (i, 0)),
            pl.BlockSpec((BM, LATENT), lambda i: (i, 0)),
            pl.BlockSpec((1, 1), lambda i: (0, 0)),
        ],
        out_shape=[
            jax.ShapeDtypeStruct((N, LATENT), jnp.float32),
            jax.ShapeDtypeStruct((N,), jnp.int32),
            jax.ShapeDtypeStruct((N, LATENT), jnp.float32),
            jax.ShapeDtypeStruct((N, LATENT), jnp.float32),
            jax.ShapeDtypeStruct((1, 1), jnp.float32),
        ],
        scratch_shapes=[pltpu.VMEM((1, K), jnp.float32)],
        compiler_params=pltpu.CompilerParams(
            dimension_semantics=("arbitrary",)),
        interpret=interpret,
    )(h, W2, b2, embedding, Wmu, bmu, Wlv, blv)


def _sc_gather(table, idx):
    b_per_w = N // SC_WORKERS
    nchunks = b_per_w // SC_CHUNK
    mesh = plsc.VectorSubcoreMesh(core_axis_name="c", subcore_axis_name="s")

    @functools.partial(
        pl.kernel,
        mesh=mesh,
        out_type=jax.ShapeDtypeStruct((N, LATENT), jnp.float32),
        scratch_types=[
            pltpu.VMEM((SC_CHUNK,), jnp.int32),
            pltpu.VMEM((SC_CHUNK, LATENT), jnp.float32),
            pltpu.SemaphoreType.DMA,
        ],
    )
    def gather_kernel(table_hbm, idx_hbm, out_hbm, idx_v, rows_v, sem):
        wid = lax.axis_index("s") * SC_CORES + lax.axis_index("c")
        base = wid * b_per_w

        def chunk(t, carry):
            off = base + t * SC_CHUNK
            pltpu.sync_copy(idx_hbm.at[pl.ds(off, SC_CHUNK)], idx_v)
            pltpu.async_copy(table_hbm.at[idx_v], rows_v, sem).wait()
            pltpu.sync_copy(rows_v, out_hbm.at[pl.ds(off, SC_CHUNK)])
            return carry

        lax.fori_loop(0, nchunks, chunk, 0)

    return gather_kernel(table, idx)


def kernel(c, W1, b1, W2, b2, embedding, Wmu, bmu, Wlv, blv):
    a = _pre_call(c, W1, b1)
    h = jax.nn.elu(a).astype(jnp.bfloat16)
    ze, idx, mu, lv, dsum = _vq_call(h, W2, b2, embedding, Wmu, bmu,
                                     Wlv, blv)
    zq = _sc_gather(embedding, idx)
    qloss = (dsum[0, 0] / (N * LATENT)).astype(jnp.float32)
    closs = COMMIT_SCALE * qloss
    return (mu, lv, zq, ze, qloss, closs)


# speedup vs baseline: 1.2246x; 1.2246x over previous
import functools

import jax
import jax.numpy as jnp
from jax import lax
from jax.experimental import pallas as pl
from jax.experimental.pallas import tpu as pltpu
from jax.experimental.pallas import tpu_sc as plsc

N = 65536
D_IN = 100
D_HID = 200
LATENT = 256
K = 8192
COMMIT_SCALE = 0.25

BM = 256
PRE_BM = 2048
CHUNK_SEGS = (
    ((0, 2688, None), (2688, 128, ('lt', 48))),
    ((2688, 128, ('ge', 48)), (2816, 2560, None), (5376, 128, ('lt', 96))),
    ((5376, 128, ('ge', 96)), (5504, 2688, None)),
)

SC_CORES = 2
SC_SUBCORES = 16
SC_WORKERS = SC_CORES * SC_SUBCORES
SC_CHUNK = 128


def _pre_body(c_ref, W1_ref, b1_ref, a_ref):
    a_ref[...] = (jnp.dot(c_ref[...], W1_ref[...],
                          preferred_element_type=jnp.float32)
                  + b1_ref[...][None, :])


def _pre_call(c, W1, b1):
    return pl.pallas_call(
        _pre_body,
        grid=(N // PRE_BM,),
        in_specs=[
            pl.BlockSpec((PRE_BM, D_IN), lambda i: (i, 0)),
            pl.BlockSpec((D_IN, D_HID), lambda i: (0, 0)),
            pl.BlockSpec((D_HID,), lambda i: (0,)),
        ],
        out_specs=pl.BlockSpec((PRE_BM, D_HID), lambda i: (i, 0)),
        out_shape=jax.ShapeDtypeStruct((N, D_HID), jnp.float32),
        compiler_params=pltpu.CompilerParams(
            dimension_semantics=("arbitrary",)),
    )(c, W1, b1)


def _s_body(E_ref, s_ref):
    e = E_ref[...]
    s_ref[...] = jnp.sum(e * e, axis=1)[None, :]


def _s_call(embedding):
    return pl.pallas_call(
        _s_body,
        out_shape=jax.ShapeDtypeStruct((1, K), jnp.float32),
    )(embedding)


def _vq_body(h_ref, W2_ref, b2_ref, E_ref, s_ref, Wmu_ref, bmu_ref,
             Wlv_ref, blv_ref, ze_ref, idx_ref, mu_ref, lv_ref, dsum_ref):
    i = pl.program_id(0)

    @pl.when(i == 0)
    def _init():
        dsum_ref[...] = jnp.zeros((1, 1), jnp.float32)

    z = jnp.dot(h_ref[...], W2_ref[...], preferred_element_type=jnp.float32) + b2_ref[...][None, :]
    ze_ref[...] = z

    r = jnp.sum(z * z, axis=1, keepdims=True)

    run_v = jnp.full((BM, 1), jnp.inf, dtype=jnp.float32)
    run_f = jnp.full((BM, 1), jnp.inf, dtype=jnp.float32)
    run_i = jnp.zeros((BM, 1), dtype=jnp.int32)
    for segs in CHUNK_SEGS:
        chunk_v = jnp.full((BM, 1), jnp.inf, dtype=jnp.float32)
        chunk_i = jnp.zeros((BM, 1), dtype=jnp.int32)
        for off, w, maskspec in segs:
            et = E_ref[pl.ds(off, w), :]
            st = s_ref[0, pl.ds(off, w)]
            m = lax.dot_general(z, et, (((1,), (1,)), ((), ())),
                                preferred_element_type=jnp.float32)
            d = (r + st[None, :]) - 2.0 * m
            iota = lax.broadcasted_iota(jnp.int32, (BM, w), 1)
            if maskspec is not None:
                kind, n = maskspec
                lane_ok = (iota < n) if kind == 'lt' else (iota >= n)
                d = jnp.where(lane_ok, d, jnp.inf)
            v = jnp.min(d, axis=1, keepdims=True)
            ti = jnp.min(jnp.where(d == v, iota, K), axis=1,
                         keepdims=True) + off
            better = v < chunk_v
            chunk_i = jnp.where(better, ti, chunk_i)
            chunk_v = jnp.where(better, v, chunk_v)
        better = chunk_v < run_v
        run_i = jnp.where(better, chunk_i, run_i)
        run_f = jnp.where(better, chunk_v, run_f)
        run_v = (jnp.where(better, chunk_v, run_v)
                 .astype(jnp.bfloat16).astype(jnp.float32))

    idx_ref[...] = run_i.reshape(BM)
    dsum_ref[...] += jnp.sum(run_f).reshape(1, 1)

    mu_ref[...] = jnp.dot(z, Wmu_ref[...], preferred_element_type=jnp.float32) + bmu_ref[...][None, :]
    lv_ref[...] = jnp.dot(z, Wlv_ref[...], preferred_element_type=jnp.float32) + blv_ref[...][None, :]


def _vq_call(h, W2, b2, embedding, s, Wmu, bmu, Wlv, blv, interpret=False):
    whole = lambda *shape: pl.BlockSpec(shape, lambda i: tuple(0 for _ in shape))
    grid = (N // BM,)
    return pl.pallas_call(
        _vq_body,
        grid=grid,
        in_specs=[
            pl.BlockSpec((BM, D_HID), lambda i: (i, 0)),
            whole(D_HID, LATENT),
            whole(LATENT),
            whole(K, LATENT),
            whole(1, K),
            whole(LATENT, LATENT),
            whole(LATENT),
            whole(LATENT, LATENT),
            whole(LATENT),
        ],
        out_specs=[
            pl.BlockSpec((BM, LATENT), lambda i: (i, 0)),
            pl.BlockSpec((BM,), lambda i: (i,)),
            pl.BlockSpec((BM, LATENT), lambda i: (i, 0)),
            pl.BlockSpec((BM, LATENT), lambda i: (i, 0)),
            pl.BlockSpec((1, 1), lambda i: (0, 0)),
        ],
        out_shape=[
            jax.ShapeDtypeStruct((N, LATENT), jnp.float32),
            jax.ShapeDtypeStruct((N,), jnp.int32),
            jax.ShapeDtypeStruct((N, LATENT), jnp.float32),
            jax.ShapeDtypeStruct((N, LATENT), jnp.float32),
            jax.ShapeDtypeStruct((1, 1), jnp.float32),
        ],
        compiler_params=pltpu.CompilerParams(
            dimension_semantics=("arbitrary",)),
        interpret=interpret,
    )(h, W2, b2, embedding, s, Wmu, bmu, Wlv, blv)


def _sc_gather(table, idx):
    b_per_w = N // SC_WORKERS
    nchunks = b_per_w // SC_CHUNK
    mesh = plsc.VectorSubcoreMesh(core_axis_name="c", subcore_axis_name="s")

    @functools.partial(
        pl.kernel,
        mesh=mesh,
        out_type=jax.ShapeDtypeStruct((N, LATENT), jnp.float32),
        scratch_types=[
            pltpu.VMEM((SC_CHUNK,), jnp.int32),
            pltpu.VMEM((SC_CHUNK, LATENT), jnp.float32),
            pltpu.SemaphoreType.DMA,
        ],
    )
    def gather_kernel(table_hbm, idx_hbm, out_hbm, idx_v, rows_v, sem):
        wid = lax.axis_index("s") * SC_CORES + lax.axis_index("c")
        base = wid * b_per_w

        def chunk(t, carry):
            off = base + t * SC_CHUNK
            pltpu.sync_copy(idx_hbm.at[pl.ds(off, SC_CHUNK)], idx_v)
            pltpu.async_copy(table_hbm.at[idx_v], rows_v, sem).wait()
            pltpu.sync_copy(rows_v, out_hbm.at[pl.ds(off, SC_CHUNK)])
            return carry

        lax.fori_loop(0, nchunks, chunk, 0)

    return gather_kernel(table, idx)


def kernel(c, W1, b1, W2, b2, embedding, Wmu, bmu, Wlv, blv):
    a = _pre_call(c, W1, b1)
    h = jax.nn.elu(a).astype(jnp.bfloat16)
    s = _s_call(embedding)
    ze, idx, mu, lv, dsum = _vq_call(h, W2, b2, embedding, s, Wmu, bmu,
                                     Wlv, blv)
    zq = _sc_gather(embedding, idx)
    qloss = (dsum[0, 0] / (N * LATENT)).astype(jnp.float32)
    closs = COMMIT_SCALE * qloss
    return (mu, lv, zq, ze, qloss, closs)


# speedup vs baseline: 1.3182x; 1.0765x over previous
import functools

import jax
import jax.numpy as jnp
from jax import lax
from jax.experimental import pallas as pl
from jax.experimental.pallas import tpu as pltpu
from jax.experimental.pallas import tpu_sc as plsc

N = 65536
D_IN = 100
D_HID = 200
LATENT = 256
K = 8192
COMMIT_SCALE = 0.25

BM = 512
PRE_BM = 2048
CHUNK_SEGS = (
    ((0, 2688, None), (2688, 128, ('lt', 48))),
    ((2688, 128, ('ge', 48)), (2816, 2560, None), (5376, 128, ('lt', 96))),
    ((5376, 128, ('ge', 96)), (5504, 2688, None)),
)

SC_CORES = 2
SC_SUBCORES = 16
SC_WORKERS = SC_CORES * SC_SUBCORES
SC_CHUNK = 128


def _pre_body(c_ref, W1_ref, b1_ref, a_ref):
    a_ref[...] = (jnp.dot(c_ref[...], W1_ref[...],
                          preferred_element_type=jnp.float32)
                  + b1_ref[...][None, :])


def _pre_call(c, W1, b1):
    return pl.pallas_call(
        _pre_body,
        grid=(N // PRE_BM,),
        in_specs=[
            pl.BlockSpec((PRE_BM, D_IN), lambda i: (i, 0)),
            pl.BlockSpec((D_IN, D_HID), lambda i: (0, 0)),
            pl.BlockSpec((D_HID,), lambda i: (0,)),
        ],
        out_specs=pl.BlockSpec((PRE_BM, D_HID), lambda i: (i, 0)),
        out_shape=jax.ShapeDtypeStruct((N, D_HID), jnp.float32),
        compiler_params=pltpu.CompilerParams(
            dimension_semantics=("arbitrary",)),
    )(c, W1, b1)


def _s_body(E_ref, s_ref):
    e = E_ref[...]
    s_ref[...] = jnp.sum(e * e, axis=1)[None, :]


def _s_call(embedding):
    return pl.pallas_call(
        _s_body,
        out_shape=jax.ShapeDtypeStruct((1, K), jnp.float32),
    )(embedding)


def _vq_body(h_ref, W2_ref, b2_ref, E_ref, s_ref, Wmu_ref, bmu_ref,
             Wlv_ref, blv_ref, ze_ref, idx_ref, mu_ref, lv_ref, dsum_ref):
    i = pl.program_id(0)

    @pl.when(i == 0)
    def _init():
        dsum_ref[...] = jnp.zeros((1, 1), jnp.float32)

    z = jnp.dot(h_ref[...], W2_ref[...], preferred_element_type=jnp.float32) + b2_ref[...][None, :]
    ze_ref[...] = z

    r = jnp.sum(z * z, axis=1, keepdims=True)

    run_v = jnp.full((BM, 1), jnp.inf, dtype=jnp.float32)
    run_f = jnp.full((BM, 1), jnp.inf, dtype=jnp.float32)
    run_i = jnp.zeros((BM, 1), dtype=jnp.int32)
    for segs in CHUNK_SEGS:
        chunk_v = jnp.full((BM, 1), jnp.inf, dtype=jnp.float32)
        chunk_i = jnp.zeros((BM, 1), dtype=jnp.int32)
        for off, w, maskspec in segs:
            et = E_ref[pl.ds(off, w), :]
            st = s_ref[0, pl.ds(off, w)]
            m = lax.dot_general(z, et, (((1,), (1,)), ((), ())),
                                preferred_element_type=jnp.float32)
            d = (r + st[None, :]) - 2.0 * m
            iota = lax.broadcasted_iota(jnp.int32, (BM, w), 1)
            if maskspec is not None:
                kind, n = maskspec
                lane_ok = (iota < n) if kind == 'lt' else (iota >= n)
                d = jnp.where(lane_ok, d, jnp.inf)
            v = jnp.min(d, axis=1, keepdims=True)
            ti = jnp.min(jnp.where(d == v, iota, K), axis=1,
                         keepdims=True) + off
            better = v < chunk_v
            chunk_i = jnp.where(better, ti, chunk_i)
            chunk_v = jnp.where(better, v, chunk_v)
        better = chunk_v < run_v
        run_i = jnp.where(better, chunk_i, run_i)
        run_f = jnp.where(better, chunk_v, run_f)
        run_v = (jnp.where(better, chunk_v, run_v)
                 .astype(jnp.bfloat16).astype(jnp.float32))

    idx_ref[...] = run_i.reshape(BM)
    dsum_ref[...] += jnp.sum(run_f).reshape(1, 1)

    mu_ref[...] = jnp.dot(z, Wmu_ref[...], preferred_element_type=jnp.float32) + bmu_ref[...][None, :]
    lv_ref[...] = jnp.dot(z, Wlv_ref[...], preferred_element_type=jnp.float32) + blv_ref[...][None, :]


def _vq_call(h, W2, b2, embedding, s, Wmu, bmu, Wlv, blv, interpret=False):
    whole = lambda *shape: pl.BlockSpec(shape, lambda i: tuple(0 for _ in shape))
    grid = (N // BM,)
    return pl.pallas_call(
        _vq_body,
        grid=grid,
        in_specs=[
            pl.BlockSpec((BM, D_HID), lambda i: (i, 0)),
            whole(D_HID, LATENT),
            whole(LATENT),
            whole(K, LATENT),
            whole(1, K),
            whole(LATENT, LATENT),
            whole(LATENT),
            whole(LATENT, LATENT),
            whole(LATENT),
        ],
        out_specs=[
            pl.BlockSpec((BM, LATENT), lambda i: (i, 0)),
            pl.BlockSpec((BM,), lambda i: (i,)),
            pl.BlockSpec((BM, LATENT), lambda i: (i, 0)),
            pl.BlockSpec((BM, LATENT), lambda i: (i, 0)),
            pl.BlockSpec((1, 1), lambda i: (0, 0)),
        ],
        out_shape=[
            jax.ShapeDtypeStruct((N, LATENT), jnp.float32),
            jax.ShapeDtypeStruct((N,), jnp.int32),
            jax.ShapeDtypeStruct((N, LATENT), jnp.float32),
            jax.ShapeDtypeStruct((N, LATENT), jnp.float32),
            jax.ShapeDtypeStruct((1, 1), jnp.float32),
        ],
        compiler_params=pltpu.CompilerParams(
            dimension_semantics=("arbitrary",)),
        interpret=interpret,
    )(h, W2, b2, embedding, s, Wmu, bmu, Wlv, blv)


def _sc_gather(table, idx):
    b_per_w = N // SC_WORKERS
    nchunks = b_per_w // SC_CHUNK
    mesh = plsc.VectorSubcoreMesh(core_axis_name="c", subcore_axis_name="s")

    @functools.partial(
        pl.kernel,
        mesh=mesh,
        out_type=jax.ShapeDtypeStruct((N, LATENT), jnp.float32),
        scratch_types=[
            pltpu.VMEM((SC_CHUNK,), jnp.int32),
            pltpu.VMEM((SC_CHUNK, LATENT), jnp.float32),
            pltpu.SemaphoreType.DMA,
        ],
    )
    def gather_kernel(table_hbm, idx_hbm, out_hbm, idx_v, rows_v, sem):
        wid = lax.axis_index("s") * SC_CORES + lax.axis_index("c")
        base = wid * b_per_w

        def chunk(t, carry):
            off = base + t * SC_CHUNK
            pltpu.sync_copy(idx_hbm.at[pl.ds(off, SC_CHUNK)], idx_v)
            pltpu.async_copy(table_hbm.at[idx_v], rows_v, sem).wait()
            pltpu.sync_copy(rows_v, out_hbm.at[pl.ds(off, SC_CHUNK)])
            return carry

        lax.fori_loop(0, nchunks, chunk, 0)

    return gather_kernel(table, idx)


def kernel(c, W1, b1, W2, b2, embedding, Wmu, bmu, Wlv, blv):
    a = _pre_call(c, W1, b1)
    h = jax.nn.elu(a).astype(jnp.bfloat16)
    s = _s_call(embedding)
    ze, idx, mu, lv, dsum = _vq_call(h, W2, b2, embedding, s, Wmu, bmu,
                                     Wlv, blv)
    zq = _sc_gather(embedding, idx)
    qloss = (dsum[0, 0] / (N * LATENT)).astype(jnp.float32)
    closs = COMMIT_SCALE * qloss
    return (mu, lv, zq, ze, qloss, closs)


# speedup vs baseline: 1.4159x; 1.0741x over previous
import functools

import jax
import jax.numpy as jnp
from jax import lax
from jax.experimental import pallas as pl
from jax.experimental.pallas import tpu as pltpu
from jax.experimental.pallas import tpu_sc as plsc

N = 65536
D_IN = 100
D_HID = 200
LATENT = 256
K = 8192
COMMIT_SCALE = 0.25

BM = 1024
PRE_BM = 2048
CHUNK_SEGS = (
    ((0, 2688, None), (2688, 128, ('lt', 48))),
    ((2688, 128, ('ge', 48)), (2816, 2560, None), (5376, 128, ('lt', 96))),
    ((5376, 128, ('ge', 96)), (5504, 2688, None)),
)

SC_CORES = 2
SC_SUBCORES = 16
SC_WORKERS = SC_CORES * SC_SUBCORES
SC_CHUNK = 128


def _pre_body(c_ref, W1_ref, b1_ref, a_ref):
    a_ref[...] = (jnp.dot(c_ref[...], W1_ref[...],
                          preferred_element_type=jnp.float32)
                  + b1_ref[...][None, :])


def _pre_call(c, W1, b1):
    return pl.pallas_call(
        _pre_body,
        grid=(N // PRE_BM,),
        in_specs=[
            pl.BlockSpec((PRE_BM, D_IN), lambda i: (i, 0)),
            pl.BlockSpec((D_IN, D_HID), lambda i: (0, 0)),
            pl.BlockSpec((D_HID,), lambda i: (0,)),
        ],
        out_specs=pl.BlockSpec((PRE_BM, D_HID), lambda i: (i, 0)),
        out_shape=jax.ShapeDtypeStruct((N, D_HID), jnp.float32),
        compiler_params=pltpu.CompilerParams(
            dimension_semantics=("arbitrary",)),
    )(c, W1, b1)


def _s_body(E_ref, s_ref):
    e = E_ref[...]
    s_ref[...] = jnp.sum(e * e, axis=1)[None, :]


def _s_call(embedding):
    return pl.pallas_call(
        _s_body,
        out_shape=jax.ShapeDtypeStruct((1, K), jnp.float32),
    )(embedding)


def _vq_body(h_ref, W2_ref, b2_ref, E_ref, s_ref, Wmu_ref, bmu_ref,
             Wlv_ref, blv_ref, ze_ref, idx_ref, mu_ref, lv_ref, dsum_ref):
    i = pl.program_id(0)

    @pl.when(i == 0)
    def _init():
        dsum_ref[...] = jnp.zeros((1, 1), jnp.float32)

    z = jnp.dot(h_ref[...], W2_ref[...], preferred_element_type=jnp.float32) + b2_ref[...][None, :]
    ze_ref[...] = z

    r = jnp.sum(z * z, axis=1, keepdims=True)

    run_v = jnp.full((BM, 1), jnp.inf, dtype=jnp.float32)
    run_f = jnp.full((BM, 1), jnp.inf, dtype=jnp.float32)
    run_i = jnp.zeros((BM, 1), dtype=jnp.int32)
    for segs in CHUNK_SEGS:
        chunk_v = jnp.full((BM, 1), jnp.inf, dtype=jnp.float32)
        chunk_i = jnp.zeros((BM, 1), dtype=jnp.int32)
        for off, w, maskspec in segs:
            et = E_ref[pl.ds(off, w), :]
            st = s_ref[0, pl.ds(off, w)]
            m = lax.dot_general(z, et, (((1,), (1,)), ((), ())),
                                preferred_element_type=jnp.float32)
            d = (r + st[None, :]) - 2.0 * m
            iota = lax.broadcasted_iota(jnp.int32, (BM, w), 1)
            if maskspec is not None:
                kind, n = maskspec
                lane_ok = (iota < n) if kind == 'lt' else (iota >= n)
                d = jnp.where(lane_ok, d, jnp.inf)
            v = jnp.min(d, axis=1, keepdims=True)
            ti = jnp.min(jnp.where(d == v, iota, K), axis=1,
                         keepdims=True) + off
            better = v < chunk_v
            chunk_i = jnp.where(better, ti, chunk_i)
            chunk_v = jnp.where(better, v, chunk_v)
        better = chunk_v < run_v
        run_i = jnp.where(better, chunk_i, run_i)
        run_f = jnp.where(better, chunk_v, run_f)
        run_v = (jnp.where(better, chunk_v, run_v)
                 .astype(jnp.bfloat16).astype(jnp.float32))

    idx_ref[...] = run_i.reshape(BM)
    dsum_ref[...] += jnp.sum(run_f).reshape(1, 1)

    mu_ref[...] = jnp.dot(z, Wmu_ref[...], preferred_element_type=jnp.float32) + bmu_ref[...][None, :]
    lv_ref[...] = jnp.dot(z, Wlv_ref[...], preferred_element_type=jnp.float32) + blv_ref[...][None, :]


def _vq_call(h, W2, b2, embedding, s, Wmu, bmu, Wlv, blv, interpret=False):
    whole = lambda *shape: pl.BlockSpec(shape, lambda i: tuple(0 for _ in shape))
    grid = (N // BM,)
    return pl.pallas_call(
        _vq_body,
        grid=grid,
        in_specs=[
            pl.BlockSpec((BM, D_HID), lambda i: (i, 0)),
            whole(D_HID, LATENT),
            whole(LATENT),
            whole(K, LATENT),
            whole(1, K),
            whole(LATENT, LATENT),
            whole(LATENT),
            whole(LATENT, LATENT),
            whole(LATENT),
        ],
        out_specs=[
            pl.BlockSpec((BM, LATENT), lambda i: (i, 0)),
            pl.BlockSpec((BM,), lambda i: (i,)),
            pl.BlockSpec((BM, LATENT), lambda i: (i, 0)),
            pl.BlockSpec((BM, LATENT), lambda i: (i, 0)),
            pl.BlockSpec((1, 1), lambda i: (0, 0)),
        ],
        out_shape=[
            jax.ShapeDtypeStruct((N, LATENT), jnp.float32),
            jax.ShapeDtypeStruct((N,), jnp.int32),
            jax.ShapeDtypeStruct((N, LATENT), jnp.float32),
            jax.ShapeDtypeStruct((N, LATENT), jnp.float32),
            jax.ShapeDtypeStruct((1, 1), jnp.float32),
        ],
        compiler_params=pltpu.CompilerParams(
            dimension_semantics=("arbitrary",)),
        interpret=interpret,
    )(h, W2, b2, embedding, s, Wmu, bmu, Wlv, blv)


def _sc_gather(table, idx):
    b_per_w = N // SC_WORKERS
    nchunks = b_per_w // SC_CHUNK
    mesh = plsc.VectorSubcoreMesh(core_axis_name="c", subcore_axis_name="s")

    @functools.partial(
        pl.kernel,
        mesh=mesh,
        out_type=jax.ShapeDtypeStruct((N, LATENT), jnp.float32),
        scratch_types=[
            pltpu.VMEM((SC_CHUNK,), jnp.int32),
            pltpu.VMEM((SC_CHUNK, LATENT), jnp.float32),
            pltpu.SemaphoreType.DMA,
        ],
    )
    def gather_kernel(table_hbm, idx_hbm, out_hbm, idx_v, rows_v, sem):
        wid = lax.axis_index("s") * SC_CORES + lax.axis_index("c")
        base = wid * b_per_w

        def chunk(t, carry):
            off = base + t * SC_CHUNK
            pltpu.sync_copy(idx_hbm.at[pl.ds(off, SC_CHUNK)], idx_v)
            pltpu.async_copy(table_hbm.at[idx_v], rows_v, sem).wait()
            pltpu.sync_copy(rows_v, out_hbm.at[pl.ds(off, SC_CHUNK)])
            return carry

        lax.fori_loop(0, nchunks, chunk, 0)

    return gather_kernel(table, idx)


def kernel(c, W1, b1, W2, b2, embedding, Wmu, bmu, Wlv, blv):
    a = _pre_call(c, W1, b1)
    h = jax.nn.elu(a).astype(jnp.bfloat16)
    s = _s_call(embedding)
    ze, idx, mu, lv, dsum = _vq_call(h, W2, b2, embedding, s, Wmu, bmu,
                                     Wlv, blv)
    zq = _sc_gather(embedding, idx)
    qloss = (dsum[0, 0] / (N * LATENT)).astype(jnp.float32)
    closs = COMMIT_SCALE * qloss
    return (mu, lv, zq, ze, qloss, closs)


# speedup vs baseline: 1.4628x; 1.0331x over previous
import functools

import jax
import jax.numpy as jnp
from jax import lax
from jax.experimental import pallas as pl
from jax.experimental.pallas import tpu as pltpu
from jax.experimental.pallas import tpu_sc as plsc

N = 65536
D_IN = 100
D_HID = 200
LATENT = 256
K = 8192
COMMIT_SCALE = 0.25

BM = 2048
PRE_BM = 2048
CHUNK_SEGS = (
    ((0, 2688, None), (2688, 128, ('lt', 48))),
    ((2688, 128, ('ge', 48)), (2816, 2560, None), (5376, 128, ('lt', 96))),
    ((5376, 128, ('ge', 96)), (5504, 2688, None)),
)

SC_CORES = 2
SC_SUBCORES = 16
SC_WORKERS = SC_CORES * SC_SUBCORES
SC_CHUNK = 128


def _pre_body(c_ref, W1_ref, b1_ref, a_ref):
    a_ref[...] = (jnp.dot(c_ref[...], W1_ref[...],
                          preferred_element_type=jnp.float32)
                  + b1_ref[...][None, :])


def _pre_call(c, W1, b1):
    return pl.pallas_call(
        _pre_body,
        grid=(N // PRE_BM,),
        in_specs=[
            pl.BlockSpec((PRE_BM, D_IN), lambda i: (i, 0)),
            pl.BlockSpec((D_IN, D_HID), lambda i: (0, 0)),
            pl.BlockSpec((D_HID,), lambda i: (0,)),
        ],
        out_specs=pl.BlockSpec((PRE_BM, D_HID), lambda i: (i, 0)),
        out_shape=jax.ShapeDtypeStruct((N, D_HID), jnp.float32),
        compiler_params=pltpu.CompilerParams(
            dimension_semantics=("arbitrary",)),
    )(c, W1, b1)


def _s_body(E_ref, s_ref):
    e = E_ref[...]
    s_ref[...] = jnp.sum(e * e, axis=1)[None, :]


def _s_call(embedding):
    return pl.pallas_call(
        _s_body,
        out_shape=jax.ShapeDtypeStruct((1, K), jnp.float32),
    )(embedding)


def _vq_body(h_ref, W2_ref, b2_ref, E_ref, s_ref, Wmu_ref, bmu_ref,
             Wlv_ref, blv_ref, ze_ref, idx_ref, mu_ref, lv_ref, dsum_ref):
    i = pl.program_id(0)

    @pl.when(i == 0)
    def _init():
        dsum_ref[...] = jnp.zeros((1, 1), jnp.float32)

    z = jnp.dot(h_ref[...], W2_ref[...], preferred_element_type=jnp.float32) + b2_ref[...][None, :]
    ze_ref[...] = z

    r = jnp.sum(z * z, axis=1, keepdims=True)

    run_v = jnp.full((BM, 1), jnp.inf, dtype=jnp.float32)
    run_f = jnp.full((BM, 1), jnp.inf, dtype=jnp.float32)
    run_i = jnp.zeros((BM, 1), dtype=jnp.int32)
    for segs in CHUNK_SEGS:
        chunk_v = jnp.full((BM, 1), jnp.inf, dtype=jnp.float32)
        chunk_i = jnp.zeros((BM, 1), dtype=jnp.int32)
        for off, w, maskspec in segs:
            et = E_ref[pl.ds(off, w), :]
            st = s_ref[0, pl.ds(off, w)]
            m = lax.dot_general(z, et, (((1,), (1,)), ((), ())),
                                preferred_element_type=jnp.float32)
            d = (r + st[None, :]) - 2.0 * m
            iota = lax.broadcasted_iota(jnp.int32, (BM, w), 1)
            if maskspec is not None:
                kind, n = maskspec
                lane_ok = (iota < n) if kind == 'lt' else (iota >= n)
                d = jnp.where(lane_ok, d, jnp.inf)
            v = jnp.min(d, axis=1, keepdims=True)
            ti = jnp.min(jnp.where(d == v, iota, K), axis=1,
                         keepdims=True) + off
            better = v < chunk_v
            chunk_i = jnp.where(better, ti, chunk_i)
            chunk_v = jnp.where(better, v, chunk_v)
        better = chunk_v < run_v
        run_i = jnp.where(better, chunk_i, run_i)
        run_f = jnp.where(better, chunk_v, run_f)
        run_v = (jnp.where(better, chunk_v, run_v)
                 .astype(jnp.bfloat16).astype(jnp.float32))

    idx_ref[...] = run_i.reshape(BM)
    dsum_ref[...] += jnp.sum(run_f).reshape(1, 1)

    mu_ref[...] = jnp.dot(z, Wmu_ref[...], preferred_element_type=jnp.float32) + bmu_ref[...][None, :]
    lv_ref[...] = jnp.dot(z, Wlv_ref[...], preferred_element_type=jnp.float32) + blv_ref[...][None, :]


def _vq_call(h, W2, b2, embedding, s, Wmu, bmu, Wlv, blv, interpret=False):
    whole = lambda *shape: pl.BlockSpec(shape, lambda i: tuple(0 for _ in shape))
    grid = (N // BM,)
    return pl.pallas_call(
        _vq_body,
        grid=grid,
        in_specs=[
            pl.BlockSpec((BM, D_HID), lambda i: (i, 0)),
            whole(D_HID, LATENT),
            whole(LATENT),
            whole(K, LATENT),
            whole(1, K),
            whole(LATENT, LATENT),
            whole(LATENT),
            whole(LATENT, LATENT),
            whole(LATENT),
        ],
        out_specs=[
            pl.BlockSpec((BM, LATENT), lambda i: (i, 0)),
            pl.BlockSpec((BM,), lambda i: (i,)),
            pl.BlockSpec((BM, LATENT), lambda i: (i, 0)),
            pl.BlockSpec((BM, LATENT), lambda i: (i, 0)),
            pl.BlockSpec((1, 1), lambda i: (0, 0)),
        ],
        out_shape=[
            jax.ShapeDtypeStruct((N, LATENT), jnp.float32),
            jax.ShapeDtypeStruct((N,), jnp.int32),
            jax.ShapeDtypeStruct((N, LATENT), jnp.float32),
            jax.ShapeDtypeStruct((N, LATENT), jnp.float32),
            jax.ShapeDtypeStruct((1, 1), jnp.float32),
        ],
        compiler_params=pltpu.CompilerParams(
            dimension_semantics=("arbitrary",)),
        interpret=interpret,
    )(h, W2, b2, embedding, s, Wmu, bmu, Wlv, blv)


def _sc_gather(table, idx):
    b_per_w = N // SC_WORKERS
    nchunks = b_per_w // SC_CHUNK
    mesh = plsc.VectorSubcoreMesh(core_axis_name="c", subcore_axis_name="s")

    @functools.partial(
        pl.kernel,
        mesh=mesh,
        out_type=jax.ShapeDtypeStruct((N, LATENT), jnp.float32),
        scratch_types=[
            pltpu.VMEM((SC_CHUNK,), jnp.int32),
            pltpu.VMEM((SC_CHUNK, LATENT), jnp.float32),
            pltpu.SemaphoreType.DMA,
        ],
    )
    def gather_kernel(table_hbm, idx_hbm, out_hbm, idx_v, rows_v, sem):
        wid = lax.axis_index("s") * SC_CORES + lax.axis_index("c")
        base = wid * b_per_w

        def chunk(t, carry):
            off = base + t * SC_CHUNK
            pltpu.sync_copy(idx_hbm.at[pl.ds(off, SC_CHUNK)], idx_v)
            pltpu.async_copy(table_hbm.at[idx_v], rows_v, sem).wait()
            pltpu.sync_copy(rows_v, out_hbm.at[pl.ds(off, SC_CHUNK)])
            return carry

        lax.fori_loop(0, nchunks, chunk, 0)

    return gather_kernel(table, idx)


def kernel(c, W1, b1, W2, b2, embedding, Wmu, bmu, Wlv, blv):
    a = _pre_call(c, W1, b1)
    h = jax.nn.elu(a).astype(jnp.bfloat16)
    s = _s_call(embedding)
    ze, idx, mu, lv, dsum = _vq_call(h, W2, b2, embedding, s, Wmu, bmu,
                                     Wlv, blv)
    zq = _sc_gather(embedding, idx)
    qloss = (dsum[0, 0] / (N * LATENT)).astype(jnp.float32)
    closs = COMMIT_SCALE * qloss
    return (mu, lv, zq, ze, qloss, closs)
